# baseline (speedup 1.0000x reference)
import dataclasses
import functools

import jax
import jax.numpy as jnp
from jax import lax
from jax.experimental import pallas as pl
from jax.experimental.pallas import tpu as pltpu
from jax.experimental.pallas import tpu_sc as plsc

D = 64
LANES = 16
NC, NS = 2, 16
NW = NC * NS
B = 16384
BW = B // NW
CHUNK = 128
NCH = BW // CHUNK

_mesh = plsc.VectorSubcoreMesh(core_axis_name="c", subcore_axis_name="s")

_cp = pltpu.CompilerParams(
    needs_layout_passes=False,
    use_tc_tiling_on_sc=False,
)


@functools.partial(
    pl.kernel,
    compiler_params=_cp,
    out_type=(
        jax.ShapeDtypeStruct((B,), jnp.float32),
        jax.ShapeDtypeStruct((B,), jnp.float32),
    ),
    mesh=_mesh,
    scratch_types=[
        pltpu.VMEM((NCH, CHUNK), jnp.int32),
        pltpu.VMEM((NCH, CHUNK), jnp.int32),
        pltpu.VMEM((NCH, CHUNK), jnp.int32),
        pltpu.VMEM((BW, D), jnp.float32),
        pltpu.VMEM((BW, D), jnp.float32),
        pltpu.VMEM((BW, D), jnp.float32),
        pltpu.VMEM((BW,), jnp.float32),
        pltpu.VMEM((BW,), jnp.float32),
        pltpu.SemaphoreType.DMA,
    ],
)
def _bpr_sc(user_table_hbm, item_table_hbm, user_hbm, item_i_hbm, item_j_hbm,
            out_i_hbm, out_j_hbm,
            idx_u, idx_i, idx_j, u_rows, i_rows, j_rows, oi, oj, sem):
    wid = lax.axis_index("s") * NC + lax.axis_index("c")
    base = wid * BW

    pltpu.sync_copy(user_hbm.at[wid], idx_u)
    pltpu.sync_copy(item_i_hbm.at[wid], idx_i)
    pltpu.sync_copy(item_j_hbm.at[wid], idx_j)

    copies = []
    for c in range(NCH):
        rows = pl.ds(c * CHUNK, CHUNK)
        copies.append(pltpu.async_copy(
            user_table_hbm.at[idx_u.at[c]], u_rows.at[rows], sem))
        copies.append(pltpu.async_copy(
            item_table_hbm.at[idx_i.at[c]], i_rows.at[rows], sem))
        copies.append(pltpu.async_copy(
            item_table_hbm.at[idx_j.at[c]], j_rows.at[rows], sem))
    for cp in copies:
        cp.wait()

    lane = lax.iota(jnp.int32, LANES)

    @pl.loop(0, BW, step=LANES)
    def _(r0):
        res_i = jnp.zeros((LANES,), jnp.float32)
        res_j = jnp.zeros((LANES,), jnp.float32)
        for rr in range(LANES):
            r = r0 + rr
            acc_i = jnp.zeros((LANES,), jnp.float32)
            acc_j = jnp.zeros((LANES,), jnp.float32)
            for c in range(D // LANES):
                cols = pl.ds(c * LANES, LANES)
                u = u_rows[r, cols]
                acc_i = acc_i + u * i_rows[r, cols]
                acc_j = acc_j + u * j_rows[r, cols]
            res_i = jnp.where(lane == rr, jnp.sum(acc_i), res_i)
            res_j = jnp.where(lane == rr, jnp.sum(acc_j), res_j)
        oi[pl.ds(r0, LANES)] = res_i
        oj[pl.ds(r0, LANES)] = res_j

    pltpu.sync_copy(oi, out_i_hbm.at[pl.ds(base, BW)])
    pltpu.sync_copy(oj, out_j_hbm.at[pl.ds(base, BW)])


def kernel(user_table, item_table, user, item_i, item_j):
    u = user.astype(jnp.int32).reshape(NW, NCH, CHUNK)
    ii = item_i.astype(jnp.int32).reshape(NW, NCH, CHUNK)
    ij = item_j.astype(jnp.int32).reshape(NW, NCH, CHUNK)
    return _bpr_sc(user_table, item_table, u, ii, ij)
